# baseline (speedup 1.0000x reference)
import functools

import jax
import jax.numpy as jnp
from jax import lax
from jax.experimental import pallas as pl
from jax.experimental.pallas import tpu as pltpu
from jax.experimental.pallas import tpu_sc as plsc

R = 64
V = 1_000_000
B = 16384
NC = 2
NS = 16
NW = NC * NS
C = B // NW
G = 16
NG = C // G


def _sc_gather(t, idx32):
    mesh = plsc.VectorSubcoreMesh(core_axis_name="c", subcore_axis_name="s")

    @functools.partial(
        pl.kernel,
        mesh=mesh,
        out_type=jax.ShapeDtypeStruct((R, B), jnp.float32),
        compiler_params=pltpu.CompilerParams(
            use_tc_tiling_on_sc=False, needs_layout_passes=False
        ),
        scratch_types=[
            pltpu.VMEM((C,), jnp.int32),
            pltpu.VMEM((G, R, G), jnp.float32),
            pltpu.VMEM((G, R, G), jnp.float32),
            pltpu.VMEM((R, C), jnp.float32),
            pltpu.SemaphoreType.DMA,
            pltpu.SemaphoreType.DMA,
        ],
    )
    def k(t_hbm, idx_hbm, out_hbm, idx_v, buf_a, buf_b, stage_v, sem_a, sem_b):
        wid = lax.axis_index("s") * NC + lax.axis_index("c")
        base = wid * C
        pltpu.sync_copy(idx_hbm.at[pl.ds(base, C)], idx_v)
        lanes = lax.iota(jnp.int32, G)

        def issue_group(g, buf, sem):
            v = idx_v[pl.ds(g * G, G)]
            for k_ in range(G):
                jb = pl.multiple_of((v[k_] >> 4) << 4, G)
                pltpu.async_copy(
                    t_hbm.at[:, pl.ds(jb, G)], buf.at[k_], sem
                )

        def wait_group(buf, sem):
            for _ in range(G):
                pltpu.make_async_copy(
                    t_hbm.at[:, pl.ds(0, G)], buf.at[0], sem
                ).wait()

        def extract_group(g, buf):
            lvec = idx_v[pl.ds(g * G, G)] & (G - 1)

            def row(r, _):
                vals = plsc.load_gather(buf, [lanes, jnp.full((G,), r, jnp.int32), lvec])
                stage_v[r, pl.ds(g * G, G)] = vals
                return 0

            lax.fori_loop(0, R, row, 0)

        issue_group(0, buf_a, sem_a)

        def body(it, _):
            issue_group(2 * it + 1, buf_b, sem_b)
            wait_group(buf_a, sem_a)
            extract_group(2 * it, buf_a)

            @pl.when(it < NG // 2 - 1)
            def _():
                issue_group(2 * it + 2, buf_a, sem_a)

            wait_group(buf_b, sem_b)
            extract_group(2 * it + 1, buf_b)
            return 0

        lax.fori_loop(0, NG // 2, body, 0)
        pltpu.sync_copy(stage_v, out_hbm.at[:, pl.ds(base, C)])

    return k(t, idx32)


def kernel(t, idx):
    return _sc_gather(t, idx.astype(jnp.int32))


# speedup vs baseline: 1.0009x; 1.0009x over previous
import functools

import jax
import jax.numpy as jnp
from jax import lax
from jax.experimental import pallas as pl
from jax.experimental.pallas import tpu as pltpu
from jax.experimental.pallas import tpu_sc as plsc

R = 64
V = 1_000_000
B = 16384
NC = 2
NS = 16
NW = NC * NS
C = B // NW
G = 16
NG = C // G


def _sc_gather(t, idx32):
    mesh = plsc.VectorSubcoreMesh(core_axis_name="c", subcore_axis_name="s")

    @functools.partial(
        pl.kernel,
        mesh=mesh,
        out_type=jax.ShapeDtypeStruct((R, B), jnp.float32),
        compiler_params=pltpu.CompilerParams(
            use_tc_tiling_on_sc=False,
            needs_layout_passes=False,
            skip_device_barrier=True,
        ),
        scratch_types=[
            pltpu.VMEM((C,), jnp.int32),
            pltpu.VMEM((G, R, G), jnp.float32),
            pltpu.VMEM((G, R, G), jnp.float32),
            pltpu.VMEM((R, C), jnp.float32),
            pltpu.SemaphoreType.DMA,
            pltpu.SemaphoreType.DMA,
        ],
    )
    def k(t_hbm, idx_hbm, out_hbm, idx_v, buf_a, buf_b, stage_v, sem_a, sem_b):
        wid = lax.axis_index("s") * NC + lax.axis_index("c")
        base = wid * C
        pltpu.sync_copy(idx_hbm.at[pl.ds(base, C)], idx_v)
        lanes = lax.iota(jnp.int32, G)

        def issue_group(g, buf, sem):
            v = idx_v[pl.ds(g * G, G)]
            for k_ in range(G):
                jb = pl.multiple_of((v[k_] >> 4) << 4, G)
                pltpu.async_copy(
                    t_hbm.at[:, pl.ds(jb, G)], buf.at[k_], sem
                )

        def wait_group(buf, sem):
            for _ in range(G):
                pltpu.make_async_copy(
                    t_hbm.at[:, pl.ds(0, G)], buf.at[0], sem
                ).wait()

        def extract_group(g, buf):
            lvec = idx_v[pl.ds(g * G, G)] & (G - 1)

            def row(r, _):
                vals = plsc.load_gather(buf, [lanes, jnp.full((G,), r, jnp.int32), lvec])
                stage_v[r, pl.ds(g * G, G)] = vals
                return 0

            lax.fori_loop(0, R, row, 0)

        issue_group(0, buf_a, sem_a)

        def body(it, _):
            issue_group(2 * it + 1, buf_b, sem_b)
            wait_group(buf_a, sem_a)
            extract_group(2 * it, buf_a)

            @pl.when(it < NG // 2 - 1)
            def _():
                issue_group(2 * it + 2, buf_a, sem_a)

            wait_group(buf_b, sem_b)
            extract_group(2 * it + 1, buf_b)
            return 0

        lax.fori_loop(0, NG // 2, body, 0)
        pltpu.sync_copy(stage_v, out_hbm.at[:, pl.ds(base, C)])

    return k(t, idx32)


def kernel(t, idx):
    return _sc_gather(t, idx.astype(jnp.int32))
